# Initial kernel scaffold; baseline (speedup 1.0000x reference)
import jax
import jax.numpy as jnp
from jax.experimental import pallas as pl


def kernel(x, edge_index, edge_attr, W_mlp0, b_mlp0, W_root0, bias0, gamma0, beta0, W_mlp1, b_mlp1, W_root1, bias1, gamma1, beta1, W_mlp2, b_mlp2, W_root2, bias2, gamma2, beta2, W_mlpf, b_mlpf, W_rootf, biasf):
    raise NotImplementedError("write your pallas kernel here")


# speedup vs baseline: 2.9173x; 2.9173x over previous
import functools

import jax
import jax.numpy as jnp
from jax import lax
from jax.experimental import pallas as pl
from jax.experimental.pallas import tpu as pltpu
from jax.experimental.pallas import tpu_sc as plsc

_N = 10000
_E = 160000
_D = 16

_NC = 2
_NS = 16
_NW = _NC * _NS
_CW = 128
_CHUNKS = 40
_EPW = _CHUNKS * _CW
_E_PAD = _NW * _EPW
_NPAD = 10240
_ZROWS = _NPAD // _NS
_OROWS = _N // _NS
_GRP = 10

@functools.lru_cache(maxsize=None)
def _sc_mesh():
    return plsc.VectorSubcoreMesh(core_axis_name="c", subcore_axis_name="s",
                                  num_cores=_NC, num_subcores=_NS)


@functools.lru_cache(maxsize=None)
def _make_sc_gather():
    return functools.partial(
        pl.kernel,
        out_type=jax.ShapeDtypeStruct((_E_PAD, _D), jnp.float32),
        mesh=_sc_mesh(),
        compiler_params=pltpu.CompilerParams(use_tc_tiling_on_sc=False),
        scratch_types=[
            pltpu.VMEM((_CHUNKS, _CW), jnp.int32),
            pltpu.VMEM((_EPW, _D), jnp.float32),
            pltpu.SemaphoreType.DMA,
        ],
    )(_gather_body)


def _sc_gather(h, src3d):
    return _make_sc_gather()(h, src3d)


def _gather_body(h_hbm, src_hbm, out_hbm, idx_v, rows_v, sem):
    wid = lax.axis_index("s") * _NC + lax.axis_index("c")
    pltpu.sync_copy(src_hbm.at[wid], idx_v)
    for g in range(_CHUNKS // _GRP):
        handles = []
        for j in range(_GRP):
            jj = g * _GRP + j
            handles.append(
                pltpu.async_copy(h_hbm.at[idx_v.at[jj]],
                                 rows_v.at[pl.ds(jj * _CW, _CW)], sem))
        for hd in handles:
            hd.wait()
    pltpu.sync_copy(rows_v, out_hbm.at[pl.ds(wid * _EPW, _EPW)])


def _scatter_body(with_deg, msg_hbm, dst_hbm, *rest):
    if with_deg:
        (out_hbm, deg_hbm, idx_v, msg_v, zbuf_v, ones_v, agg_sh, deg_sh,
         sem) = rest
    else:
        out_hbm, idx_v, msg_v, zbuf_v, agg_sh, sem = rest
        deg_hbm = ones_v = deg_sh = None
    cid = lax.axis_index("c")
    sid = lax.axis_index("s")
    wid = sid * _NC + cid

    @pl.loop(0, 64)
    def _(i):
        zbuf_v[i, :] = jnp.zeros((_D,), jnp.float32)

    @pl.loop(0, _ZROWS // 64)
    def _(i):
        pltpu.sync_copy(zbuf_v, agg_sh.at[pl.ds(sid * _ZROWS + i * 64, 64)])

    if with_deg:
        @pl.loop(0, _CW)
        def _(i):
            ones_v[i, :] = jnp.ones((_D,), jnp.float32)

        @pl.loop(0, _ZROWS // 64)
        def _(i):
            pltpu.sync_copy(zbuf_v, deg_sh.at[pl.ds(sid * _ZROWS + i * 64, 64)])

    plsc.subcore_barrier()

    pltpu.sync_copy(dst_hbm.at[wid], idx_v)
    pltpu.sync_copy(msg_hbm.at[pl.ds(wid * _EPW, _EPW)], msg_v)
    for g in range(_CHUNKS // _GRP):
        handles = []
        for j in range(_GRP):
            jj = g * _GRP + j
            handles.append(
                pltpu.async_copy(msg_v.at[pl.ds(jj * _CW, _CW)],
                                 agg_sh.at[idx_v.at[jj]], sem, add=True))
            if with_deg:
                handles.append(
                    pltpu.async_copy(ones_v, deg_sh.at[idx_v.at[jj]], sem,
                                     add=True))
        for hd in handles:
            hd.wait()

    plsc.subcore_barrier()

    pltpu.sync_copy(agg_sh.at[pl.ds(sid * _OROWS, _OROWS)],
                    out_hbm.at[cid, pl.ds(sid * _OROWS, _OROWS)])
    if with_deg:
        pltpu.sync_copy(deg_sh.at[pl.ds(sid * _OROWS, _OROWS)],
                        deg_hbm.at[cid, pl.ds(sid * _OROWS, _OROWS)])


@functools.lru_cache(maxsize=None)
def _make_sc_scatter(with_deg):
    if with_deg:
        out_type = (jax.ShapeDtypeStruct((_NC, _N, _D), jnp.float32),
                    jax.ShapeDtypeStruct((_NC, _N, _D), jnp.float32))
        scratch = [
            pltpu.VMEM((_CHUNKS, _CW), jnp.int32),
            pltpu.VMEM((_EPW, _D), jnp.float32),
            pltpu.VMEM((64, _D), jnp.float32),
            pltpu.VMEM((_CW, _D), jnp.float32),
            pltpu.VMEM_SHARED((_NPAD, _D), jnp.float32),
            pltpu.VMEM_SHARED((_NPAD, _D), jnp.float32),
            pltpu.SemaphoreType.DMA,
        ]
    else:
        out_type = jax.ShapeDtypeStruct((_NC, _N, _D), jnp.float32)
        scratch = [
            pltpu.VMEM((_CHUNKS, _CW), jnp.int32),
            pltpu.VMEM((_EPW, _D), jnp.float32),
            pltpu.VMEM((64, _D), jnp.float32),
            pltpu.VMEM_SHARED((_NPAD, _D), jnp.float32),
            pltpu.SemaphoreType.DMA,
        ]
    return functools.partial(
        pl.kernel, out_type=out_type, mesh=_sc_mesh(),
        compiler_params=pltpu.CompilerParams(use_tc_tiling_on_sc=False),
        scratch_types=scratch,
    )(functools.partial(_scatter_body, with_deg))


def _sc_scatter_deg(msg, dst3d):
    return _make_sc_scatter(True)(msg, dst3d)


def _sc_scatter(msg, dst3d):
    return _make_sc_scatter(False)(msg, dst3d)


_BE = 2048


def _edge_conv_body(ea_ref, xg_ref, w_ref, b_ref, r_ref, s_ref, out_ref):
    th = jnp.tanh(
        jnp.dot(ea_ref[...], w_ref[...], preferred_element_type=jnp.float32)
        + b_ref[...])
    xe = jnp.dot(xg_ref[...], r_ref[...], preferred_element_type=jnp.float32)
    out_ref[...] = jnp.dot(xe * th, s_ref[...],
                           preferred_element_type=jnp.float32)


def _edge_final_body(ea_ref, xg_ref, w_ref, b_ref, ones_ref, unused_ref,
                     out_ref):
    th = jnp.tanh(
        jnp.dot(ea_ref[...], w_ref[...], preferred_element_type=jnp.float32)
        + b_ref[...])
    out_ref[...] = jnp.dot(xg_ref[...] * th, ones_ref[...],
                           preferred_element_type=jnp.float32)


def _tc_edges(ea, xg, w, b, m1, m2, final):
    k = w.shape[1]
    body = _edge_final_body if final else _edge_conv_body
    return pl.pallas_call(
        body,
        grid=(_E_PAD // _BE,),
        in_specs=[
            pl.BlockSpec((_BE, _D), lambda i: (i, 0)),
            pl.BlockSpec((_BE, _D), lambda i: (i, 0)),
            pl.BlockSpec((_D, k), lambda i: (0, 0)),
            pl.BlockSpec((1, k), lambda i: (0, 0)),
            pl.BlockSpec(m1.shape, lambda i: (0, 0)),
            pl.BlockSpec(m2.shape, lambda i: (0, 0)),
        ],
        out_specs=pl.BlockSpec((_BE, _D), lambda i: (i, 0)),
        out_shape=jax.ShapeDtypeStruct((_E_PAD, _D), jnp.float32),
    )(ea, xg, w, b, m1, m2)


def _update0_body(h_ref, ap_ref, dp_ref, wr_ref, b_ref, g_ref, be_ref,
                  out_ref, inv_ref):
    inv = 1.0 / jnp.maximum(dp_ref[0] + dp_ref[1], 1.0)
    inv_ref[...] = inv
    _update_common(h_ref, ap_ref, inv, wr_ref, b_ref, g_ref, be_ref, out_ref)


def _update_body(h_ref, ap_ref, inv_ref, wr_ref, b_ref, g_ref, be_ref,
                 out_ref):
    _update_common(h_ref, ap_ref, inv_ref[...], wr_ref, b_ref, g_ref, be_ref,
                   out_ref)


def _update_common(h_ref, ap_ref, inv, wr_ref, b_ref, g_ref, be_ref, out_ref):
    h = h_ref[...]
    agg = (ap_ref[0] + ap_ref[1]) * inv
    t = jnp.dot(h, wr_ref[...], preferred_element_type=jnp.float32) + agg \
        + b_ref[...]
    m = jnp.mean(t, axis=0, keepdims=True)
    v = jnp.mean((t - m) * (t - m), axis=0, keepdims=True)
    t = (t - m) * lax.rsqrt(v + 1e-5) * g_ref[...] + be_ref[...]
    out_ref[...] = h + jnp.maximum(t, 0.0)


def _final_body(h_ref, ap_ref, inv_ref, wr_ref, b_ref, out_ref):
    agg = (ap_ref[0] + ap_ref[1]) * inv_ref[...]
    out_ref[...] = jnp.dot(h_ref[...], wr_ref[...],
                           preferred_element_type=jnp.float32) + agg \
        + b_ref[...]


def _tc_update0(h, ap, dp, wr, b, g, be):
    return pl.pallas_call(
        _update0_body,
        out_shape=(jax.ShapeDtypeStruct((_N, _D), jnp.float32),
                   jax.ShapeDtypeStruct((_N, _D), jnp.float32)),
    )(h, ap, dp, wr, b, g, be)


def _tc_update(h, ap, inv, wr, b, g, be):
    return pl.pallas_call(
        _update_body,
        out_shape=jax.ShapeDtypeStruct((_N, _D), jnp.float32),
    )(h, ap, inv, wr, b, g, be)


def _tc_final(h, ap, inv, wr, b):
    return pl.pallas_call(
        _final_body,
        out_shape=jax.ShapeDtypeStruct((_N, _D), jnp.float32),
    )(h, ap, inv, wr, b)


def kernel(x, edge_index, edge_attr, W_mlp0, b_mlp0, W_root0, bias0, gamma0,
           beta0, W_mlp1, b_mlp1, W_root1, bias1, gamma1, beta1, W_mlp2,
           b_mlp2, W_root2, bias2, gamma2, beta2, W_mlpf, b_mlpf, W_rootf,
           biasf):
    src = edge_index[0]
    dst = edge_index[1]
    src3d = jnp.pad(src, (0, _E_PAD - _E)).reshape(_NW, _CHUNKS, _CW)
    dst3d = jnp.pad(dst, (0, _E_PAD - _E),
                    constant_values=_N).reshape(_NW, _CHUNKS, _CW)
    ea = jnp.pad(edge_attr, ((0, _E_PAD - _E), (0, 0)))

    ids = jnp.arange(_D * _D, dtype=jnp.int32)
    expand = (ids[None, :] // _D == jnp.arange(_D, dtype=jnp.int32)[:, None]
              ).astype(jnp.float32)
    collapse = (ids[:, None] % _D == jnp.arange(_D, dtype=jnp.int32)[None, :]
                ).astype(jnp.float32)
    ones_dd = jnp.ones((_D, _D), jnp.float32)

    h = x
    inv = None
    layers = [(W_mlp0, b_mlp0, W_root0, bias0, gamma0, beta0),
              (W_mlp1, b_mlp1, W_root1, bias1, gamma1, beta1),
              (W_mlp2, b_mlp2, W_root2, bias2, gamma2, beta2)]
    for li, (wm, bm, wr, bb, g, be) in enumerate(layers):
        xg = _sc_gather(h, src3d)
        msg = _tc_edges(ea, xg, wm, bm.reshape(1, -1), expand, collapse,
                        final=False)
        if li == 0:
            ap, dp = _sc_scatter_deg(msg, dst3d)
            h, inv = _tc_update0(h, ap, dp, wr, bb.reshape(1, _D),
                                 g.reshape(1, _D), be.reshape(1, _D))
        else:
            ap = _sc_scatter(msg, dst3d)
            h = _tc_update(h, ap, inv, wr, bb.reshape(1, _D),
                           g.reshape(1, _D), be.reshape(1, _D))

    xg = _sc_gather(h, src3d)
    msgf = _tc_edges(ea, xg, W_mlpf, b_mlpf.reshape(1, _D), ones_dd, ones_dd,
                     final=True)
    apf = _sc_scatter(msgf, dst3d)
    wrf16 = W_rootf @ jnp.ones((1, _D), jnp.float32)
    bf16 = jnp.broadcast_to(biasf.reshape(1, 1), (1, _D))
    out16 = _tc_final(h, apf, inv, wrf16, bf16)
    return out16[:, :1]


# speedup vs baseline: 5.6551x; 1.9385x over previous
import functools

import jax
import jax.numpy as jnp
from jax import lax
from jax.experimental import pallas as pl
from jax.experimental.pallas import tpu as pltpu
from jax.experimental.pallas import tpu_sc as plsc

_N = 10000
_E = 160000
_D = 16

_NC = 2
_NS = 16
_NW = _NC * _NS
_CW = 128
_CHUNKS = 40
_EPW = _CHUNKS * _CW
_E_PAD = _NW * _EPW
_NPAD = 10240
_ZROWS = _NPAD // _NS
_OROWS = _N // _NS
_GRP = 10

@functools.lru_cache(maxsize=None)
def _sc_mesh():
    return plsc.VectorSubcoreMesh(core_axis_name="c", subcore_axis_name="s",
                                  num_cores=_NC, num_subcores=_NS)


@functools.lru_cache(maxsize=None)
def _make_sc_gather():
    return functools.partial(
        pl.kernel,
        out_type=jax.ShapeDtypeStruct((_E_PAD, _D), jnp.float32),
        mesh=_sc_mesh(),
        compiler_params=pltpu.CompilerParams(use_tc_tiling_on_sc=False),
        scratch_types=[
            pltpu.VMEM((_CHUNKS, _CW), jnp.int32),
            pltpu.VMEM((_EPW, _D), jnp.float32),
            pltpu.SemaphoreType.DMA,
        ],
    )(_gather_body)


def _sc_gather(h, src3d):
    return _make_sc_gather()(h, src3d)


def _gather_body(h_hbm, src_hbm, out_hbm, idx_v, rows_v, sem):
    wid = lax.axis_index("s") * _NC + lax.axis_index("c")
    pltpu.sync_copy(src_hbm.at[wid], idx_v)
    for g in range(_CHUNKS // _GRP):
        handles = []
        for j in range(_GRP):
            jj = g * _GRP + j
            handles.append(
                pltpu.async_copy(h_hbm.at[idx_v.at[jj]],
                                 rows_v.at[pl.ds(jj * _CW, _CW)], sem))
        for hd in handles:
            hd.wait()
    pltpu.sync_copy(rows_v, out_hbm.at[pl.ds(wid * _EPW, _EPW)])


def _scatter_body(with_deg, msg_hbm, dst_hbm, *rest):
    if with_deg:
        (out_hbm, deg_hbm, idx_v, msg_v, zbuf_v, ones_v, agg_sh, deg_sh,
         sem) = rest
    else:
        out_hbm, idx_v, msg_v, zbuf_v, agg_sh, sem = rest
        deg_hbm = ones_v = deg_sh = None
    cid = lax.axis_index("c")
    sid = lax.axis_index("s")
    wid = sid * _NC + cid

    @pl.loop(0, 64)
    def _(i):
        zbuf_v[i, :] = jnp.zeros((_D,), jnp.float32)

    @pl.loop(0, _ZROWS // 64)
    def _(i):
        pltpu.sync_copy(zbuf_v, agg_sh.at[pl.ds(sid * _ZROWS + i * 64, 64)])

    if with_deg:
        @pl.loop(0, _CW)
        def _(i):
            ones_v[i, :] = jnp.ones((_D,), jnp.float32)

        @pl.loop(0, _ZROWS // 64)
        def _(i):
            pltpu.sync_copy(zbuf_v, deg_sh.at[pl.ds(sid * _ZROWS + i * 64, 64)])

    plsc.subcore_barrier()

    pltpu.sync_copy(dst_hbm.at[wid], idx_v)
    pltpu.sync_copy(msg_hbm.at[pl.ds(wid * _EPW, _EPW)], msg_v)
    for g in range(_CHUNKS // _GRP):
        handles = []
        for j in range(_GRP):
            jj = g * _GRP + j
            handles.append(
                pltpu.async_copy(msg_v.at[pl.ds(jj * _CW, _CW)],
                                 agg_sh.at[idx_v.at[jj]], sem, add=True))
            if with_deg:
                handles.append(
                    pltpu.async_copy(ones_v, deg_sh.at[idx_v.at[jj]], sem,
                                     add=True))
        for hd in handles:
            hd.wait()

    plsc.subcore_barrier()

    pltpu.sync_copy(agg_sh.at[pl.ds(sid * _OROWS, _OROWS)],
                    out_hbm.at[cid, pl.ds(sid * _OROWS, _OROWS)])
    if with_deg:
        pltpu.sync_copy(deg_sh.at[pl.ds(sid * _OROWS, _OROWS)],
                        deg_hbm.at[cid, pl.ds(sid * _OROWS, _OROWS)])


@functools.lru_cache(maxsize=None)
def _make_sc_scatter(with_deg):
    if with_deg:
        out_type = (jax.ShapeDtypeStruct((_NC, _N, _D), jnp.float32),
                    jax.ShapeDtypeStruct((_NC, _N, _D), jnp.float32))
        scratch = [
            pltpu.VMEM((_CHUNKS, _CW), jnp.int32),
            pltpu.VMEM((_EPW, _D), jnp.float32),
            pltpu.VMEM((64, _D), jnp.float32),
            pltpu.VMEM((_CW, _D), jnp.float32),
            pltpu.VMEM_SHARED((_NPAD, _D), jnp.float32),
            pltpu.VMEM_SHARED((_NPAD, _D), jnp.float32),
            pltpu.SemaphoreType.DMA,
        ]
    else:
        out_type = jax.ShapeDtypeStruct((_NC, _N, _D), jnp.float32)
        scratch = [
            pltpu.VMEM((_CHUNKS, _CW), jnp.int32),
            pltpu.VMEM((_EPW, _D), jnp.float32),
            pltpu.VMEM((64, _D), jnp.float32),
            pltpu.VMEM_SHARED((_NPAD, _D), jnp.float32),
            pltpu.SemaphoreType.DMA,
        ]
    return functools.partial(
        pl.kernel, out_type=out_type, mesh=_sc_mesh(),
        compiler_params=pltpu.CompilerParams(use_tc_tiling_on_sc=False),
        scratch_types=scratch,
    )(functools.partial(_scatter_body, with_deg))


def _sc_scatter_deg(msg, dst3d):
    return _make_sc_scatter(True)(msg, dst3d)


def _sc_scatter(msg, dst3d):
    return _make_sc_scatter(False)(msg, dst3d)


_BE = 2048


def _edge_conv_body(ea_ref, xg_ref, w_ref, b_ref, r_ref, s_ref, out_ref):
    acc = jnp.zeros((_BE, 128), jnp.float32)
    for k in range(8):
        th = jnp.tanh(
            jnp.dot(ea_ref[...], w_ref[k],
                    preferred_element_type=jnp.float32) + b_ref[...])
        xe = jnp.dot(xg_ref[...], r_ref[k],
                     preferred_element_type=jnp.float32)
        acc = acc + jnp.dot(xe * th, s_ref[k],
                            preferred_element_type=jnp.float32)
    out_ref[...] = acc


def _edge_final_body(ea_ref, xg_ref, w_ref, b_ref, p_ref, bk_ref, out_ref):
    acc = jnp.zeros((_BE, 128), jnp.float32)
    for k in range(8):
        th = jnp.tanh(
            jnp.dot(ea_ref[...], w_ref[k],
                    preferred_element_type=jnp.float32) + b_ref[...])
        xk = jnp.dot(xg_ref[...], p_ref[k],
                     preferred_element_type=jnp.float32)
        acc = acc + jnp.dot(xk * th, bk_ref[k],
                            preferred_element_type=jnp.float32)
    out_ref[...] = acc


def _tc_edges(ea8, xg8, wstack, b, m1, m2, final):
    body = _edge_final_body if final else _edge_conv_body
    nrows = _E_PAD // 8
    return pl.pallas_call(
        body,
        grid=(nrows // _BE,),
        in_specs=[
            pl.BlockSpec((_BE, 128), lambda i: (i, 0)),
            pl.BlockSpec((_BE, 128), lambda i: (i, 0)),
            pl.BlockSpec(wstack.shape, lambda i: (0, 0, 0)),
            pl.BlockSpec(b.shape, lambda i: (0, 0)),
            pl.BlockSpec(m1.shape, lambda i: (0, 0, 0)),
            pl.BlockSpec(m2.shape, lambda i: (0, 0, 0)),
        ],
        out_specs=pl.BlockSpec((_BE, 128), lambda i: (i, 0)),
        out_shape=jax.ShapeDtypeStruct((nrows, 128), jnp.float32),
    )(ea8, xg8, wstack, b, m1, m2)


def _update0_body(h_ref, ap_ref, dp_ref, wr_ref, b_ref, g_ref, be_ref,
                  out_ref, inv_ref):
    inv = 1.0 / jnp.maximum(dp_ref[0] + dp_ref[1], 1.0)
    inv_ref[...] = inv
    _update_common(h_ref, ap_ref, inv, wr_ref, b_ref, g_ref, be_ref, out_ref)


def _update_body(h_ref, ap_ref, inv_ref, wr_ref, b_ref, g_ref, be_ref,
                 out_ref):
    _update_common(h_ref, ap_ref, inv_ref[...], wr_ref, b_ref, g_ref, be_ref,
                   out_ref)


def _update_common(h_ref, ap_ref, inv, wr_ref, b_ref, g_ref, be_ref, out_ref):
    h = h_ref[...]
    agg = (ap_ref[0] + ap_ref[1]) * inv
    t = jnp.dot(h, wr_ref[...], preferred_element_type=jnp.float32) + agg \
        + b_ref[...]
    m = jnp.mean(t, axis=0, keepdims=True)
    v = jnp.mean((t - m) * (t - m), axis=0, keepdims=True)
    t = (t - m) * lax.rsqrt(v + 1e-5) * g_ref[...] + be_ref[...]
    out_ref[...] = h + jnp.maximum(t, 0.0)


def _final_body(h_ref, ap_ref, inv_ref, wr_ref, b_ref, out_ref):
    agg = (ap_ref[0] + ap_ref[1]) * inv_ref[...]
    out_ref[...] = jnp.dot(h_ref[...], wr_ref[...],
                           preferred_element_type=jnp.float32) + agg \
        + b_ref[...]


def _tc_update0(h, ap, dp, wr, b, g, be):
    return pl.pallas_call(
        _update0_body,
        out_shape=(jax.ShapeDtypeStruct((_N, _D), jnp.float32),
                   jax.ShapeDtypeStruct((_N, _D), jnp.float32)),
    )(h, ap, dp, wr, b, g, be)


def _tc_update(h, ap, inv, wr, b, g, be):
    return pl.pallas_call(
        _update_body,
        out_shape=jax.ShapeDtypeStruct((_N, _D), jnp.float32),
    )(h, ap, inv, wr, b, g, be)


def _tc_final(h, ap, inv, wr, b):
    return pl.pallas_call(
        _final_body,
        out_shape=jax.ShapeDtypeStruct((_N, _D), jnp.float32),
    )(h, ap, inv, wr, b)


def kernel(x, edge_index, edge_attr, W_mlp0, b_mlp0, W_root0, bias0, gamma0,
           beta0, W_mlp1, b_mlp1, W_root1, bias1, gamma1, beta1, W_mlp2,
           b_mlp2, W_root2, bias2, gamma2, beta2, W_mlpf, b_mlpf, W_rootf,
           biasf):
    src = edge_index[0]
    dst = edge_index[1]
    src3d = jnp.pad(src, (0, _E_PAD - _E)).reshape(_NW, _CHUNKS, _CW)
    dst3d = jnp.pad(dst, (0, _E_PAD - _E),
                    constant_values=_N).reshape(_NW, _CHUNKS, _CW)
    ea8 = jnp.pad(edge_attr.reshape(_E // 8, 128), ((0, (_E_PAD - _E) // 8),
                                                    (0, 0)))

    ids = jnp.arange(_D * _D, dtype=jnp.int32)
    expand = (ids[None, :] // _D == jnp.arange(_D, dtype=jnp.int32)[:, None]
              ).astype(jnp.float32)
    collapse = (ids[:, None] % _D == jnp.arange(_D, dtype=jnp.int32)[None, :]
                ).astype(jnp.float32)
    sel = (jnp.arange(128, dtype=jnp.int32)[None, :, None] ==
           16 * jnp.arange(8, dtype=jnp.int32)[:, None, None]
           + jnp.arange(_D, dtype=jnp.int32)[None, None, :]
           ).astype(jnp.float32)

    h = x
    inv = None
    layers = [(W_mlp0, b_mlp0, W_root0, bias0, gamma0, beta0),
              (W_mlp1, b_mlp1, W_root1, bias1, gamma1, beta1),
              (W_mlp2, b_mlp2, W_root2, bias2, gamma2, beta2)]
    for li, (wm, bm, wr, bb, g, be) in enumerate(layers):
        wstack = jnp.einsum('kci,ij->kcj', sel, wm)
        rstack = jnp.einsum('kci,ij->kcj', sel, expand)
        sstack = jnp.einsum('jo,kco->kjc', collapse, sel)
        xg = _sc_gather(h, src3d)
        xg8 = xg.reshape(_E_PAD // 8, 128)
        msg8 = _tc_edges(ea8, xg8, wstack, bm.reshape(1, -1), rstack, sstack,
                         final=False)
        msg = msg8.reshape(_E_PAD, _D)
        if li == 0:
            ap, dp = _sc_scatter_deg(msg, dst3d)
            h, inv = _tc_update0(h, ap, dp, wr, bb.reshape(1, _D),
                                 g.reshape(1, _D), be.reshape(1, _D))
        else:
            ap = _sc_scatter(msg, dst3d)
            h = _tc_update(h, ap, inv, wr, bb.reshape(1, _D),
                           g.reshape(1, _D), be.reshape(1, _D))

    wfstack = jnp.einsum('kci,ij->kcj', sel, W_mlpf)
    bstack = jnp.broadcast_to(sel.sum(-1)[:, None, :], (8, _D, 128))
    xg = _sc_gather(h, src3d)
    xg8 = xg.reshape(_E_PAD // 8, 128)
    msgf8 = _tc_edges(ea8, xg8, wfstack, b_mlpf.reshape(1, _D), sel, bstack,
                      final=True)
    apf = _sc_scatter(msgf8.reshape(_E_PAD, _D), dst3d)
    wrf16 = W_rootf @ jnp.ones((1, _D), jnp.float32)
    bf16 = jnp.broadcast_to(biasf.reshape(1, 1), (1, _D))
    out16 = _tc_final(h, apf, inv, wrf16, bf16)
    return out16[:, :1]


# speedup vs baseline: 7.5530x; 1.3356x over previous
import functools

import jax
import jax.numpy as jnp
from jax import lax
from jax.experimental import pallas as pl
from jax.experimental.pallas import tpu as pltpu
from jax.experimental.pallas import tpu_sc as plsc

_N = 10000
_E = 160000
_D = 16

_NC = 2
_NS = 16
_NW = _NC * _NS
_CW = 128
_CHUNKS = 40
_EPW = _CHUNKS * _CW
_E_PAD = _NW * _EPW
_NPAD = 10240
_ZROWS = _NPAD // _NS
_OROWS = _N // _NS
_GRP = 10

@functools.lru_cache(maxsize=None)
def _sc_mesh():
    return plsc.VectorSubcoreMesh(core_axis_name="c", subcore_axis_name="s",
                                  num_cores=_NC, num_subcores=_NS)


@functools.lru_cache(maxsize=None)
def _make_sc_gather():
    return functools.partial(
        pl.kernel,
        out_type=jax.ShapeDtypeStruct((_E_PAD, _D), jnp.float32),
        mesh=_sc_mesh(),
        compiler_params=pltpu.CompilerParams(use_tc_tiling_on_sc=False),
        scratch_types=[
            pltpu.VMEM((_CHUNKS, _CW), jnp.int32),
            pltpu.VMEM((_EPW, _D), jnp.float32),
            pltpu.VMEM_SHARED((_N, _D), jnp.float32),
            pltpu.SemaphoreType.DMA,
        ],
    )(_gather_body)


def _sc_gather(h, src3d):
    return _make_sc_gather()(h, src3d)


def _gather_body(h_hbm, src_hbm, out_hbm, idx_v, rows_v, h_sh, sem):
    sid = lax.axis_index("s")
    wid = sid * _NC + lax.axis_index("c")
    pltpu.sync_copy(h_hbm.at[pl.ds(sid * _OROWS, _OROWS)],
                    h_sh.at[pl.ds(sid * _OROWS, _OROWS)])
    pltpu.sync_copy(src_hbm.at[wid], idx_v)
    plsc.subcore_barrier()
    for g in range(_CHUNKS // _GRP):
        handles = []
        for j in range(_GRP):
            jj = g * _GRP + j
            handles.append(
                pltpu.async_copy(h_sh.at[idx_v.at[jj]],
                                 rows_v.at[pl.ds(jj * _CW, _CW)], sem))
        for hd in handles:
            hd.wait()
    pltpu.sync_copy(rows_v, out_hbm.at[pl.ds(wid * _EPW, _EPW)])


def _scatter_body(with_deg, msg_hbm, dst_hbm, *rest):
    if with_deg:
        (out_hbm, deg_hbm, idx_v, msg_v, zbuf_v, ones_v, agg_sh, deg_sh,
         sem) = rest
    else:
        out_hbm, idx_v, msg_v, zbuf_v, agg_sh, sem = rest
        deg_hbm = ones_v = deg_sh = None
    cid = lax.axis_index("c")
    sid = lax.axis_index("s")
    wid = sid * _NC + cid

    @pl.loop(0, 64)
    def _(i):
        zbuf_v[i, :] = jnp.zeros((_D,), jnp.float32)

    @pl.loop(0, _ZROWS // 64)
    def _(i):
        pltpu.sync_copy(zbuf_v, agg_sh.at[pl.ds(sid * _ZROWS + i * 64, 64)])

    if with_deg:
        @pl.loop(0, _CW)
        def _(i):
            ones_v[i, :] = jnp.ones((_D,), jnp.float32)

        @pl.loop(0, _ZROWS // 64)
        def _(i):
            pltpu.sync_copy(zbuf_v, deg_sh.at[pl.ds(sid * _ZROWS + i * 64, 64)])

    plsc.subcore_barrier()

    pltpu.sync_copy(dst_hbm.at[wid], idx_v)
    pltpu.sync_copy(msg_hbm.at[pl.ds(wid * _EPW, _EPW)], msg_v)
    for g in range(_CHUNKS // _GRP):
        handles = []
        for j in range(_GRP):
            jj = g * _GRP + j
            handles.append(
                pltpu.async_copy(msg_v.at[pl.ds(jj * _CW, _CW)],
                                 agg_sh.at[idx_v.at[jj]], sem, add=True))
            if with_deg:
                handles.append(
                    pltpu.async_copy(ones_v, deg_sh.at[idx_v.at[jj]], sem,
                                     add=True))
        for hd in handles:
            hd.wait()

    plsc.subcore_barrier()

    pltpu.sync_copy(agg_sh.at[pl.ds(sid * _ZROWS, _ZROWS)],
                    out_hbm.at[cid, pl.ds(sid * _ZROWS, _ZROWS)])
    if with_deg:
        pltpu.sync_copy(deg_sh.at[pl.ds(sid * _ZROWS, _ZROWS)],
                        deg_hbm.at[cid, pl.ds(sid * _ZROWS, _ZROWS)])


@functools.lru_cache(maxsize=None)
def _make_sc_scatter(with_deg):
    if with_deg:
        out_type = (jax.ShapeDtypeStruct((_NC, _NPAD, _D), jnp.float32),
                    jax.ShapeDtypeStruct((_NC, _NPAD, _D), jnp.float32))
        scratch = [
            pltpu.VMEM((_CHUNKS, _CW), jnp.int32),
            pltpu.VMEM((_EPW, _D), jnp.float32),
            pltpu.VMEM((64, _D), jnp.float32),
            pltpu.VMEM((_CW, _D), jnp.float32),
            pltpu.VMEM_SHARED((_NPAD, _D), jnp.float32),
            pltpu.VMEM_SHARED((_NPAD, _D), jnp.float32),
            pltpu.SemaphoreType.DMA,
        ]
    else:
        out_type = jax.ShapeDtypeStruct((_NC, _NPAD, _D), jnp.float32)
        scratch = [
            pltpu.VMEM((_CHUNKS, _CW), jnp.int32),
            pltpu.VMEM((_EPW, _D), jnp.float32),
            pltpu.VMEM((64, _D), jnp.float32),
            pltpu.VMEM_SHARED((_NPAD, _D), jnp.float32),
            pltpu.SemaphoreType.DMA,
        ]
    return functools.partial(
        pl.kernel, out_type=out_type, mesh=_sc_mesh(),
        compiler_params=pltpu.CompilerParams(use_tc_tiling_on_sc=False),
        scratch_types=scratch,
    )(functools.partial(_scatter_body, with_deg))


def _sc_scatter_deg(msg, dst3d):
    return _make_sc_scatter(True)(msg, dst3d)


def _sc_scatter(msg, dst3d):
    return _make_sc_scatter(False)(msg, dst3d)


_BE = 2048


def _edge_conv_body(ea_ref, xg_ref, w_ref, b_ref, r_ref, s_ref, out_ref):
    acc = jnp.zeros((_BE, 128), jnp.float32)
    for k in range(8):
        th = jnp.tanh(
            jnp.dot(ea_ref[...], w_ref[k],
                    preferred_element_type=jnp.float32) + b_ref[...])
        xe = jnp.dot(xg_ref[...], r_ref[k],
                     preferred_element_type=jnp.float32)
        acc = acc + jnp.dot(xe * th, s_ref[k],
                            preferred_element_type=jnp.float32)
    out_ref[...] = acc


def _edge_final_body(ea_ref, xg_ref, w_ref, b_ref, p_ref, bk_ref, out_ref):
    acc = jnp.zeros((_BE, 128), jnp.float32)
    for k in range(8):
        th = jnp.tanh(
            jnp.dot(ea_ref[...], w_ref[k],
                    preferred_element_type=jnp.float32) + b_ref[...])
        xk = jnp.dot(xg_ref[...], p_ref[k],
                     preferred_element_type=jnp.float32)
        acc = acc + jnp.dot(xk * th, bk_ref[k],
                            preferred_element_type=jnp.float32)
    out_ref[...] = acc


def _tc_edges(ea8, xg8, wstack, b, m1, m2, final):
    body = _edge_final_body if final else _edge_conv_body
    nrows = _E_PAD // 8
    return pl.pallas_call(
        body,
        grid=(nrows // _BE,),
        in_specs=[
            pl.BlockSpec((_BE, 128), lambda i: (i, 0)),
            pl.BlockSpec((_BE, 128), lambda i: (i, 0)),
            pl.BlockSpec(wstack.shape, lambda i: (0, 0, 0)),
            pl.BlockSpec(b.shape, lambda i: (0, 0)),
            pl.BlockSpec(m1.shape, lambda i: (0, 0, 0)),
            pl.BlockSpec(m2.shape, lambda i: (0, 0, 0)),
        ],
        out_specs=pl.BlockSpec((_BE, 128), lambda i: (i, 0)),
        out_shape=jax.ShapeDtypeStruct((nrows, 128), jnp.float32),
    )(ea8, xg8, wstack, b, m1, m2)


_NR = _N * _D // 128
_NPR = _NPAD * _D // 128


def _update0_body(h_ref, ap_ref, dp_ref, wr_ref, fold_ref, b_ref, g_ref,
                  be_ref, out_ref, inv_ref):
    inv = 1.0 / jnp.maximum(dp_ref[0:_NPR] + dp_ref[_NPR:2 * _NPR], 1.0)
    inv_ref[...] = inv
    _update_common(h_ref, ap_ref, inv[0:_NR], wr_ref, fold_ref, b_ref, g_ref,
                   be_ref, out_ref)


def _update_body(h_ref, ap_ref, inv_ref, wr_ref, fold_ref, b_ref, g_ref,
                 be_ref, out_ref):
    _update_common(h_ref, ap_ref, inv_ref[0:_NR], wr_ref, fold_ref, b_ref,
                   g_ref, be_ref, out_ref)


def _update_common(h_ref, ap_ref, inv, wr_ref, fold_ref, b_ref, g_ref, be_ref,
                   out_ref):
    h = h_ref[...]
    agg = (ap_ref[0:_NR] + ap_ref[_NPR:_NPR + _NR]) * inv
    t = jnp.dot(h, wr_ref[...], preferred_element_type=jnp.float32) + agg \
        + b_ref[...]
    csum = jnp.sum(t, axis=0, keepdims=True)
    m = jnp.dot(csum, fold_ref[...], preferred_element_type=jnp.float32) \
        * (1.0 / _N)
    s2 = jnp.sum(t * t, axis=0, keepdims=True)
    ex2 = jnp.dot(s2, fold_ref[...], preferred_element_type=jnp.float32) \
        * (1.0 / _N)
    v = ex2 - m * m
    t = (t - m) * lax.rsqrt(v + 1e-5) * g_ref[...] + be_ref[...]
    out_ref[...] = h + jnp.maximum(t, 0.0)


def _final_body(h_ref, ap_ref, inv_ref, wr_ref, b_ref, out_ref):
    agg = (ap_ref[0:_NR] + ap_ref[_NPR:_NPR + _NR]) * inv_ref[0:_NR]
    out_ref[...] = jnp.dot(h_ref[...], wr_ref[...],
                           preferred_element_type=jnp.float32) + agg \
        + b_ref[...]


def _tc_update0(h8, ap8, dp8, wrbd, fold, b, g, be):
    return pl.pallas_call(
        _update0_body,
        out_shape=(jax.ShapeDtypeStruct((_NR, 128), jnp.float32),
                   jax.ShapeDtypeStruct((_NPR, 128), jnp.float32)),
    )(h8, ap8, dp8, wrbd, fold, b, g, be)


def _tc_update(h8, ap8, inv8, wrbd, fold, b, g, be):
    return pl.pallas_call(
        _update_body,
        out_shape=jax.ShapeDtypeStruct((_NR, 128), jnp.float32),
    )(h8, ap8, inv8, wrbd, fold, b, g, be)


def _tc_final(h8, ap8, inv8, wrbd, b):
    return pl.pallas_call(
        _final_body,
        out_shape=jax.ShapeDtypeStruct((_NR, 128), jnp.float32),
    )(h8, ap8, inv8, wrbd, b)


def kernel(x, edge_index, edge_attr, W_mlp0, b_mlp0, W_root0, bias0, gamma0,
           beta0, W_mlp1, b_mlp1, W_root1, bias1, gamma1, beta1, W_mlp2,
           b_mlp2, W_root2, bias2, gamma2, beta2, W_mlpf, b_mlpf, W_rootf,
           biasf):
    src = edge_index[0]
    dst = edge_index[1]
    src3d = jnp.pad(src, (0, _E_PAD - _E)).reshape(_NW, _CHUNKS, _CW)
    dst3d = jnp.pad(dst, (0, _E_PAD - _E),
                    constant_values=_N).reshape(_NW, _CHUNKS, _CW)
    ea8 = jnp.pad(edge_attr.reshape(_E // 8, 128), ((0, (_E_PAD - _E) // 8),
                                                    (0, 0)))

    ids = jnp.arange(_D * _D, dtype=jnp.int32)
    expand = (ids[None, :] // _D == jnp.arange(_D, dtype=jnp.int32)[:, None]
              ).astype(jnp.float32)
    collapse = (ids[:, None] % _D == jnp.arange(_D, dtype=jnp.int32)[None, :]
                ).astype(jnp.float32)
    sel = (jnp.arange(128, dtype=jnp.int32)[None, :, None] ==
           16 * jnp.arange(8, dtype=jnp.int32)[:, None, None]
           + jnp.arange(_D, dtype=jnp.int32)[None, None, :]
           ).astype(jnp.float32)

    fold = (jnp.arange(128, dtype=jnp.int32)[:, None] % _D ==
            jnp.arange(128, dtype=jnp.int32)[None, :] % _D
            ).astype(jnp.float32)

    def tile16(vec):
        return jnp.tile(vec.reshape(1, _D), (1, 8))

    h8 = x.reshape(_NR, 128)
    inv8 = None
    layers = [(W_mlp0, b_mlp0, W_root0, bias0, gamma0, beta0),
              (W_mlp1, b_mlp1, W_root1, bias1, gamma1, beta1),
              (W_mlp2, b_mlp2, W_root2, bias2, gamma2, beta2)]
    for li, (wm, bm, wr, bb, g, be) in enumerate(layers):
        wstack = jnp.einsum('kci,ij->kcj', sel, wm)
        rstack = jnp.einsum('kci,ij->kcj', sel, expand)
        sstack = jnp.einsum('jo,kco->kjc', collapse, sel)
        wrbd = jnp.einsum('kci,ij,kdj->cd', sel, wr, sel)
        xg = _sc_gather(h8.reshape(_N, _D), src3d)
        xg8 = xg.reshape(_E_PAD // 8, 128)
        msg8 = _tc_edges(ea8, xg8, wstack, bm.reshape(1, -1), rstack, sstack,
                         final=False)
        msg = msg8.reshape(_E_PAD, _D)
        if li == 0:
            ap, dp = _sc_scatter_deg(msg, dst3d)
            h8, inv8 = _tc_update0(h8, ap.reshape(2 * _NPR, 128),
                                   dp.reshape(2 * _NPR, 128), wrbd, fold,
                                   tile16(bb), tile16(g), tile16(be))
        else:
            ap = _sc_scatter(msg, dst3d)
            h8 = _tc_update(h8, ap.reshape(2 * _NPR, 128), inv8, wrbd, fold,
                            tile16(bb), tile16(g), tile16(be))

    wfstack = jnp.einsum('kci,ij->kcj', sel, W_mlpf)
    bstack = jnp.broadcast_to(sel.sum(-1)[:, None, :], (8, _D, 128))
    xg = _sc_gather(h8.reshape(_N, _D), src3d)
    xg8 = xg.reshape(_E_PAD // 8, 128)
    msgf8 = _tc_edges(ea8, xg8, wfstack, b_mlpf.reshape(1, _D), sel, bstack,
                      final=True)
    apf = _sc_scatter(msgf8.reshape(_E_PAD, _D), dst3d)
    wrf16 = W_rootf @ jnp.ones((1, _D), jnp.float32)
    wrfbd = jnp.einsum('kci,ij,kdj->cd', sel, wrf16, sel)
    bf128 = jnp.broadcast_to(biasf.reshape(1, 1), (1, 128))
    out8 = _tc_final(h8, apf.reshape(2 * _NPR, 128), inv8, wrfbd, bf128)
    return out8.reshape(_N, _D)[:, :1]
